# baseline (speedup 1.0000x reference)
import functools

import jax
import jax.numpy as jnp
import numpy as np
from jax import lax
from jax.experimental import pallas as pl
from jax.experimental.pallas import tpu as pltpu
from jax.experimental.pallas import tpu_sc as plsc

LANES = 16


def _pos_encoding_np(length: int, d_model: int) -> np.ndarray:
    position = np.arange(0, length, dtype=np.float32)[:, None]
    i2 = np.arange(0, d_model, step=2).astype(np.float32)
    emb = np.zeros((length, d_model), dtype=np.float32)
    emb[:, 0::2] = np.sin(position / 10000 ** (i2 / d_model))
    emb[:, 1::2] = np.cos(position / 10000 ** (i2 / d_model))
    return emb


@functools.lru_cache(maxsize=None)
def _pos_const(length: int, d_model: int):
    return jnp.asarray(_pos_encoding_np(length, d_model))


def _sc_info():
    try:
        info = plsc.get_sparse_core_info()
        return info.num_cores, info.num_subcores
    except Exception:
        return 2, 16


@functools.lru_cache(maxsize=None)
def _build(B: int, L: int, D: int):
    NC, NS = _sc_info()
    NW = NC * NS
    assert L % NW == 0
    cols = L // NW
    CH = 32
    assert cols % CH == 0
    n_chunks_per_b = cols // CH
    nvec = D // LANES

    mesh = plsc.VectorSubcoreMesh(core_axis_name="c", subcore_axis_name="s")

    NBUF = 3
    chunks = [(b, c * CH) for b in range(B) for c in range(n_chunks_per_b)]
    N = len(chunks)

    @functools.partial(
        pl.kernel,
        mesh=mesh,
        out_type=jax.ShapeDtypeStruct((B, L, D), jnp.float32),
        scratch_types=[
            pltpu.VMEM((N, CH), jnp.int32),
            pltpu.VMEM((cols, D), jnp.float32),
            pltpu.VMEM((NBUF, CH, D), jnp.float32),
            pltpu.SemaphoreType.DMA,
            pltpu.SemaphoreType.DMA,
            pltpu.SemaphoreType.DMA,
            pltpu.SemaphoreType.DMA,
            pltpu.SemaphoreType.DMA,
            pltpu.SemaphoreType.DMA,
            pltpu.SemaphoreType.DMA,
        ],
    )
    def k(x_hbm, table_hbm, pos_hbm, out_hbm, idx_v, pos_v, bufs,
          g0, g1, g2, w0, w1, w2, psem):
        gsems = (g0, g1, g2)
        wsems = (w0, w1, w2)
        wid = lax.axis_index("s") * NC + lax.axis_index("c")
        l0 = wid * cols
        for i, (b, off) in enumerate(chunks):
            pltpu.sync_copy(x_hbm.at[b, pl.ds(l0 + off, CH)], idx_v.at[i])
        pos_cp = pltpu.async_copy(pos_hbm.at[pl.ds(l0, cols)], pos_v, psem)

        def gather(i):
            s = i % NBUF
            return pltpu.async_copy(
                table_hbm.at[idx_v.at[i]], bufs.at[s], gsems[s]
            )

        g = [None] * N
        w = [None] * N
        g[0] = gather(0)
        for i in range(N):
            s = i % NBUF
            b, off = chunks[i]
            if i + 1 < N:
                if i + 1 >= NBUF:
                    w[i + 1 - NBUF].wait()
                g[i + 1] = gather(i + 1)
            g[i].wait()
            if i == 0:
                pos_cp.wait()

            def row_body(r, _, off=off, s=s):
                for j in range(nvec):
                    sl = pl.ds(j * LANES, LANES)
                    plsc.addupdate(bufs.at[s, r, sl], pos_v[off + r, sl])
                return 0

            lax.fori_loop(0, CH, row_body, 0)
            w[i] = pltpu.async_copy(
                bufs.at[s], out_hbm.at[b, pl.ds(l0 + off, CH)], wsems[s]
            )
        for i in range(max(0, N - NBUF), N):
            w[i].wait()

    return k


def kernel(x, table):
    B, L = x.shape
    D = table.shape[1]
    pos = _pos_const(L, D)
    return _build(B, L, D)(x, table, pos)


# speedup vs baseline: 1.0164x; 1.0164x over previous
import functools

import jax
import jax.numpy as jnp
import numpy as np
from jax import lax
from jax.experimental import pallas as pl
from jax.experimental.pallas import tpu as pltpu
from jax.experimental.pallas import tpu_sc as plsc

LANES = 16
NCHUNK = 2
CH = 32
NBUF = 3
BR = 512


def _pos_encoding_np(length: int, d_model: int) -> np.ndarray:
    position = np.arange(0, length, dtype=np.float32)[:, None]
    i2 = np.arange(0, d_model, step=2).astype(np.float32)
    emb = np.zeros((length, d_model), dtype=np.float32)
    emb[:, 0::2] = np.sin(position / 10000 ** (i2 / d_model))
    emb[:, 1::2] = np.cos(position / 10000 ** (i2 / d_model))
    return emb


@functools.lru_cache(maxsize=None)
def _pos_const(length: int, d_model: int):
    return jnp.asarray(_pos_encoding_np(length, d_model))


def _sc_info():
    try:
        info = plsc.get_sparse_core_info()
        return info.num_cores, info.num_subcores
    except Exception:
        return 2, 16


@functools.lru_cache(maxsize=None)
def _build_gather(T: int, V: int, D: int):
    NC, NS = _sc_info()
    NW = NC * NS
    assert T % (NW * CH) == 0
    pw = T // NW
    n = pw // CH

    mesh = plsc.VectorSubcoreMesh(core_axis_name="c", subcore_axis_name="s")

    @functools.partial(
        pl.kernel,
        mesh=mesh,
        out_type=jax.ShapeDtypeStruct((T, D), jnp.float32),
        scratch_types=[
            pltpu.VMEM((n, CH), jnp.int32),
            pltpu.VMEM((NBUF, CH, D), jnp.float32),
        ] + [pltpu.SemaphoreType.DMA] * (2 * NBUF),
    )
    def k(xc_hbm, table_hbm, out_hbm, idx_v, bufs, *sems):
        gsems = sems[:NBUF]
        wsems = sems[NBUF:]
        wid = lax.axis_index("s") * NC + lax.axis_index("c")
        base = wid * pw
        for i in range(n):
            pltpu.sync_copy(xc_hbm.at[pl.ds(base + i * CH, CH)], idx_v.at[i])

        def gather(i):
            s = i % NBUF
            return pltpu.async_copy(
                table_hbm.at[idx_v.at[i]], bufs.at[s], gsems[s]
            )

        g = [None] * n
        w = [None] * n
        g[0] = gather(0)
        for i in range(n):
            s = i % NBUF
            if i + 1 < n:
                if i + 1 >= NBUF:
                    w[i + 1 - NBUF].wait()
                g[i + 1] = gather(i + 1)
            g[i].wait()
            w[i] = pltpu.async_copy(
                bufs.at[s], out_hbm.at[pl.ds(base + i * CH, CH)], wsems[s]
            )
        for i in range(max(0, n - NBUF), n):
            w[i].wait()

    return k


@functools.lru_cache(maxsize=None)
def _build_add(TOT: int, T: int, L: int, D: int, row0: int, first: bool):
    nsteps = T // BR
    pos_period = L // BR

    def body(*refs):
        g_ref, p_ref, o_ref = refs[-3:]
        o_ref[...] = g_ref[...] + p_ref[...]

    in_specs = [
        pl.BlockSpec((BR, D), lambda i: (i, 0)),
        pl.BlockSpec((BR, D), lambda i: ((row0 // BR + i) % pos_period, 0)),
    ]
    kwargs = {}
    if not first:
        in_specs = [pl.BlockSpec(memory_space=pl.ANY)] + in_specs
        kwargs["input_output_aliases"] = {0: 0}

    return pl.pallas_call(
        body,
        grid=(nsteps,),
        in_specs=in_specs,
        out_specs=pl.BlockSpec((BR, D), lambda i: (row0 // BR + i, 0)),
        out_shape=jax.ShapeDtypeStruct((TOT, D), jnp.float32),
        **kwargs,
    )


def kernel(x, table):
    B, L = x.shape
    V, D = table.shape
    TOT = B * L
    T = TOT // NCHUNK
    pos = _pos_const(L, D)
    xf = x.reshape(TOT)

    sc_gather = _build_gather(T, V, D)
    gathered = [sc_gather(xf[k * T:(k + 1) * T], table) for k in range(NCHUNK)]

    out = _build_add(TOT, T, L, D, 0, True)(gathered[0], pos)
    for k in range(1, NCHUNK):
        out = _build_add(TOT, T, L, D, k * T, False)(out, gathered[k], pos)
    return out.reshape(B, L, D)


# speedup vs baseline: 1.1072x; 1.0893x over previous
import functools

import jax
import jax.numpy as jnp
import numpy as np
from jax import lax
from jax.experimental import pallas as pl
from jax.experimental.pallas import tpu as pltpu
from jax.experimental.pallas import tpu_sc as plsc

LANES = 16


def _pos_encoding_np(length: int, d_model: int) -> np.ndarray:
    position = np.arange(0, length, dtype=np.float32)[:, None]
    i2 = np.arange(0, d_model, step=2).astype(np.float32)
    emb = np.zeros((length, d_model), dtype=np.float32)
    emb[:, 0::2] = np.sin(position / 10000 ** (i2 / d_model))
    emb[:, 1::2] = np.cos(position / 10000 ** (i2 / d_model))
    return emb


@functools.lru_cache(maxsize=None)
def _pos_const(length: int, d_model: int):
    return jnp.asarray(_pos_encoding_np(length, d_model))


def _sc_info():
    try:
        info = plsc.get_sparse_core_info()
        return info.num_cores, info.num_subcores
    except Exception:
        return 2, 16


@functools.lru_cache(maxsize=None)
def _build(B: int, L: int, D: int):
    NC, NS = _sc_info()
    NW = NC * NS
    assert L % NW == 0
    cols = L // NW
    CH = 32
    assert cols % CH == 0
    n_chunks_per_b = cols // CH
    nvec = D // LANES

    mesh = plsc.VectorSubcoreMesh(core_axis_name="c", subcore_axis_name="s")

    NBUF = 3
    chunks = [(b, c * CH) for b in range(B) for c in range(n_chunks_per_b)]
    N = len(chunks)

    @functools.partial(
        pl.kernel,
        mesh=mesh,
        out_type=jax.ShapeDtypeStruct((B, L, D), jnp.float32),
        scratch_types=[
            pltpu.VMEM((N, CH), jnp.int32),
            pltpu.VMEM((cols, D), jnp.float32),
            pltpu.VMEM((NBUF, CH, D), jnp.float32),
            pltpu.SemaphoreType.DMA,
            pltpu.SemaphoreType.DMA,
            pltpu.SemaphoreType.DMA,
            pltpu.SemaphoreType.DMA,
            pltpu.SemaphoreType.DMA,
            pltpu.SemaphoreType.DMA,
            pltpu.SemaphoreType.DMA,
        ],
    )
    def k(x_hbm, table_hbm, pos_hbm, out_hbm, idx_v, pos_v, bufs,
          g0, g1, g2, w0, w1, w2, psem):
        gsems = (g0, g1, g2)
        wsems = (w0, w1, w2)
        wid = lax.axis_index("s") * NC + lax.axis_index("c")
        l0 = wid * cols
        for i, (b, off) in enumerate(chunks):
            pltpu.sync_copy(x_hbm.at[b, pl.ds(l0 + off, CH)], idx_v.at[i])
        pos_cp = pltpu.async_copy(pos_hbm.at[pl.ds(l0, cols)], pos_v, psem)

        def gather(i):
            s = i % NBUF
            return pltpu.async_copy(
                table_hbm.at[idx_v.at[i]], bufs.at[s], gsems[s]
            )

        g = [None] * N
        w = [None] * N
        g[0] = gather(0)
        for i in range(N):
            s = i % NBUF
            b, off = chunks[i]
            if i + 1 < N:
                if i + 1 >= NBUF:
                    w[i + 1 - NBUF].wait()
                g[i + 1] = gather(i + 1)
            g[i].wait()
            if i == 0:
                pos_cp.wait()

            @plsc.parallel_loop(0, CH, 1, unroll=4)
            def row_body(r, off=off, s=s):
                for j in range(nvec):
                    sl = pl.ds(j * LANES, LANES)
                    plsc.addupdate(bufs.at[s, r, sl], pos_v[off + r, sl])
            w[i] = pltpu.async_copy(
                bufs.at[s], out_hbm.at[b, pl.ds(l0 + off, CH)], wsems[s]
            )
        for i in range(max(0, N - NBUF), N):
            w[i].wait()

    return k


def kernel(x, table):
    B, L = x.shape
    D = table.shape[1]
    pos = _pos_const(L, D)
    return _build(B, L, D)(x, table, pos)


# speedup vs baseline: 1.2339x; 1.1144x over previous
import functools

import jax
import jax.numpy as jnp
import numpy as np
from jax import lax
from jax.experimental import pallas as pl
from jax.experimental.pallas import tpu as pltpu
from jax.experimental.pallas import tpu_sc as plsc

LANES = 16


def _pos_encoding_np(length: int, d_model: int) -> np.ndarray:
    position = np.arange(0, length, dtype=np.float32)[:, None]
    i2 = np.arange(0, d_model, step=2).astype(np.float32)
    emb = np.zeros((length, d_model), dtype=np.float32)
    emb[:, 0::2] = np.sin(position / 10000 ** (i2 / d_model))
    emb[:, 1::2] = np.cos(position / 10000 ** (i2 / d_model))
    return emb


@functools.lru_cache(maxsize=None)
def _pos_const(length: int, d_model: int):
    return jnp.asarray(_pos_encoding_np(length, d_model))


def _sc_info():
    try:
        info = plsc.get_sparse_core_info()
        return info.num_cores, info.num_subcores
    except Exception:
        return 2, 16


@functools.lru_cache(maxsize=None)
def _build(B: int, L: int, D: int):
    NC, NS = _sc_info()
    NW = NC * NS
    assert L % NW == 0
    cols = L // NW
    CH = 32
    assert cols % CH == 0
    n_chunks_per_b = cols // CH
    nvec = D // LANES

    mesh = plsc.VectorSubcoreMesh(core_axis_name="c", subcore_axis_name="s")

    NBUF = 3
    chunks = [(b, c * CH) for b in range(B) for c in range(n_chunks_per_b)]
    N = len(chunks)

    @functools.partial(
        pl.kernel,
        mesh=mesh,
        out_type=jax.ShapeDtypeStruct((B, L, D), jnp.float32),
        scratch_types=[
            pltpu.VMEM((B, cols), jnp.int32),
            pltpu.VMEM((cols, D), jnp.float32),
            pltpu.VMEM((NBUF, CH, D), jnp.float32),
            pltpu.SemaphoreType.DMA,
            pltpu.SemaphoreType.DMA,
            pltpu.SemaphoreType.DMA,
            pltpu.SemaphoreType.DMA,
            pltpu.SemaphoreType.DMA,
            pltpu.SemaphoreType.DMA,
            pltpu.SemaphoreType.DMA,
        ],
    )
    def k(x_hbm, table_hbm, pos_hbm, out_hbm, idx_v, pos_v, bufs,
          g0, g1, g2, w0, w1, w2, psem):
        gsems = (g0, g1, g2)
        wsems = (w0, w1, w2)
        wid = lax.axis_index("s") * NC + lax.axis_index("c")
        l0 = wid * cols
        for b in range(B):
            pltpu.sync_copy(x_hbm.at[b, pl.ds(l0, cols)], idx_v.at[b])
        pos_cp = pltpu.async_copy(pos_hbm.at[pl.ds(l0, cols)], pos_v, psem)

        def gather(i):
            b, off = chunks[i]
            s = i % NBUF
            return pltpu.async_copy(
                table_hbm.at[idx_v.at[b, pl.ds(off, CH)]], bufs.at[s],
                gsems[s]
            )

        g = [None] * N
        w = [None] * N
        g[0] = gather(0)
        for i in range(N):
            s = i % NBUF
            b, off = chunks[i]
            if i + 1 < N:
                if i + 1 >= NBUF:
                    w[i + 1 - NBUF].wait()
                g[i + 1] = gather(i + 1)
            g[i].wait()
            if i == 0:
                pos_cp.wait()

            @plsc.parallel_loop(0, CH, 1, unroll=2)
            def row_body(r, off=off, s=s):
                for j in range(nvec):
                    sl = pl.ds(j * LANES, LANES)
                    plsc.addupdate(bufs.at[s, r, sl], pos_v[off + r, sl])
            w[i] = pltpu.async_copy(
                bufs.at[s], out_hbm.at[b, pl.ds(l0 + off, CH)], wsems[s]
            )
        for i in range(max(0, N - NBUF), N):
            w[i].wait()

    return k


def kernel(x, table):
    B, L = x.shape
    D = table.shape[1]
    pos = _pos_const(L, D)
    return _build(B, L, D)(x, table, pos)


# speedup vs baseline: 1.2704x; 1.0295x over previous
import functools

import jax
import jax.numpy as jnp
import numpy as np
from jax import lax
from jax.experimental import pallas as pl
from jax.experimental.pallas import tpu as pltpu
from jax.experimental.pallas import tpu_sc as plsc

LANES = 16


def _pos_encoding_np(length: int, d_model: int) -> np.ndarray:
    position = np.arange(0, length, dtype=np.float32)[:, None]
    i2 = np.arange(0, d_model, step=2).astype(np.float32)
    emb = np.zeros((length, d_model), dtype=np.float32)
    emb[:, 0::2] = np.sin(position / 10000 ** (i2 / d_model))
    emb[:, 1::2] = np.cos(position / 10000 ** (i2 / d_model))
    return emb


@functools.lru_cache(maxsize=None)
def _pos_const(length: int, d_model: int):
    return jnp.asarray(_pos_encoding_np(length, d_model))


def _sc_info():
    try:
        info = plsc.get_sparse_core_info()
        return info.num_cores, info.num_subcores
    except Exception:
        return 2, 16


@functools.lru_cache(maxsize=None)
def _build(B: int, L: int, D: int):
    NC, NS = _sc_info()
    NW = NC * NS
    assert L % NW == 0
    cols = L // NW
    CH = 32
    assert cols % CH == 0
    n_chunks_per_b = cols // CH
    nvec = D // LANES

    mesh = plsc.VectorSubcoreMesh(core_axis_name="c", subcore_axis_name="s")

    NBUF = 3
    chunks = [(b, c * CH) for b in range(B) for c in range(n_chunks_per_b)]
    N = len(chunks)

    @functools.partial(
        pl.kernel,
        mesh=mesh,
        out_type=jax.ShapeDtypeStruct((B, L, D), jnp.float32),
        scratch_types=[
            pltpu.VMEM((B, cols), jnp.int32),
            pltpu.VMEM((cols, D), jnp.float32),
            pltpu.VMEM((NBUF, CH, D), jnp.float32),
            pltpu.SemaphoreType.DMA,
            pltpu.SemaphoreType.DMA,
            pltpu.SemaphoreType.DMA,
            pltpu.SemaphoreType.DMA,
            pltpu.SemaphoreType.DMA,
            pltpu.SemaphoreType.DMA,
            pltpu.SemaphoreType.DMA,
            pltpu.SemaphoreType.DMA,
        ],
    )
    def k(x_hbm, table_hbm, pos_hbm, out_hbm, idx_v, pos_v, bufs,
          g0, g1, g2, w0, w1, w2, psem, isem):
        gsems = (g0, g1, g2)
        wsems = (w0, w1, w2)
        wid = lax.axis_index("s") * NC + lax.axis_index("c")
        l0 = wid * cols
        icps = [
            pltpu.async_copy(x_hbm.at[b, pl.ds(l0, cols)], idx_v.at[b], isem)
            for b in range(B)
        ]
        pos_cp = pltpu.async_copy(pos_hbm.at[pl.ds(l0, cols)], pos_v, psem)
        for cp in icps:
            cp.wait()

        def gather(i):
            b, off = chunks[i]
            s = i % NBUF
            return pltpu.async_copy(
                table_hbm.at[idx_v.at[b, pl.ds(off, CH)]], bufs.at[s],
                gsems[s]
            )

        g = [None] * N
        w = [None] * N
        g[0] = gather(0)
        for i in range(N):
            s = i % NBUF
            b, off = chunks[i]
            if i + 1 < N:
                if i + 1 >= NBUF:
                    w[i + 1 - NBUF].wait()
                g[i + 1] = gather(i + 1)
            g[i].wait()
            if i == 0:
                pos_cp.wait()

            @plsc.parallel_loop(0, CH, 1, unroll=2)
            def row_body(r, off=off, s=s):
                for j in range(nvec):
                    sl = pl.ds(j * LANES, LANES)
                    plsc.addupdate(bufs.at[s, r, sl], pos_v[off + r, sl])
            w[i] = pltpu.async_copy(
                bufs.at[s], out_hbm.at[b, pl.ds(l0 + off, CH)], wsems[s]
            )
        for i in range(max(0, N - NBUF), N):
            w[i].wait()

    return k


def kernel(x, table):
    B, L = x.shape
    D = table.shape[1]
    pos = _pos_const(L, D)
    return _build(B, L, D)(x, table, pos)
